# baseline (speedup 1.0000x reference)
import jax
import jax.numpy as jnp
from jax import lax
from jax.experimental import pallas as pl
from jax.experimental.pallas import tpu as pltpu
from jax.experimental.pallas import tpu_sc as plsc

_NC = 2
_NS = 16
_L = 16

_V = 1000000
_D = 64
_B = 16384

_P = 1024
_MAXR = 16
_RW = 2048
_NBKT = 31
_WIN = 16
_DUMP = _B

_TAIL_A0 = 999424
_TAIL_R0 = 999936

_NROWS1 = 32 * _MAXR * _P * _D


def _rsqrt16(x):
    i = plsc.bitcast(x, jnp.int32)
    y = plsc.bitcast(jnp.int32(0x5F3759DF) - (i >> 1), jnp.float32)
    for _ in range(3):
        y = y * (1.5 - (0.5 * x) * y * y)
    return y


def _ceil16(n):
    return (n + 15) >> 4


_MESH = dict(core_axis_name="c", subcore_axis_name="s",
             num_cores=_NC, num_subcores=_NS)


def _build_k1():
    mesh = plsc.VectorSubcoreMesh(**_MESH)

    def body(ids_hbm, tab_hbm, tail_hbm, rows1_hbm, bs_hbm, meta_hbm,
             idx_v, pr_v, pb_v, sub_r, sub_k, so_v, chunk_v, rows_v, meta_v,
             tail_v, sem):
        w = lax.axis_index("s") * _NC + lax.axis_index("c")
        lanes = lax.iota(jnp.int32, _L)
        base = w << 15

        def scan_ids(per_vreg):
            def outer(ci, o):
                pltpu.sync_copy(ids_hbm.at[pl.ds(ci * 4096, 4096)], idx_v)

                def blk(i, o):
                    rvec = idx_v[pl.ds(i * _L, _L)]
                    bvec = ci * 4096 + i * _L + lanes
                    return per_vreg(bvec, rvec, o)

                return lax.fori_loop(0, 256, blk, o)

            return lax.fori_loop(0, 4, outer, jnp.int32(0))

        def bucket_of(rvec):
            return jnp.minimum(rvec >> 15, jnp.int32(_NBKT - 1))

        def count_vreg(bvec, rvec, o):
            m = bucket_of(rvec) == w
            return o + plsc.all_reduce_population_count(m)[0]

        count = scan_ids(count_vreg)
        rounds = (count + (_P - 1)) >> 10

        def round_body(t, carry):
            lo = t * _P
            for j in range(_P // _L):
                pr_v[pl.ds(j * _L, _L)] = jnp.full((_L,), 0, jnp.int32) + base
                pb_v[pl.ds(j * _L, _L)] = jnp.full((_L,), _DUMP, jnp.int32)
            pr_v[pl.ds(_P, _L)] = jnp.full((_L,), 0, jnp.int32) + base
            pb_v[pl.ds(_P, _L)] = jnp.full((_L,), _DUMP, jnp.int32)

            def fill_vreg(bvec, rvec, o):
                m = bucket_of(rvec) == w
                cnt = plsc.all_reduce_population_count(m)[0]
                rank = o + plsc.cumsum(m.astype(jnp.int32)) - 1
                sel = m & (rank >= lo) & (rank < lo + _P)
                dst = jnp.minimum(jnp.maximum(o - lo, 0), _P)
                plsc.store_compressed(pr_v.at[pl.ds(dst, _L)], rvec, mask=sel)
                plsc.store_compressed(pb_v.at[pl.ds(dst, _L)], bvec, mask=sel)
                return o + cnt

            scan_ids(fill_vreg)

            o2 = jnp.int32(0)
            for win in range(_WIN):
                so_v[pl.ds(win * _L, _L)] = jnp.zeros((_L,), jnp.int32) + o2

                def sblk(i, o2, win=win):
                    rvec = pr_v[pl.ds(i * _L, _L)]
                    kvec = i * _L + lanes
                    m = ((rvec - base) >> 11) == win
                    cnt = plsc.all_reduce_population_count(m)[0]
                    plsc.store_compressed(sub_r.at[pl.ds(o2, _L)], rvec, mask=m)
                    plsc.store_compressed(sub_k.at[pl.ds(o2, _L)], kvec, mask=m)
                    return o2 + cnt

                o2 = lax.fori_loop(0, _P // _L, sblk, o2)
            so_v[pl.ds(_WIN * _L, _L)] = jnp.zeros((_L,), jnp.int32) + o2

            def seg(win):
                s0 = so_v[pl.ds(win * _L, _L)][0]
                s1 = so_v[pl.ds(win * _L + _L, _L)][0]
                return s0, s1

            def extract(d0, win, r0, width, lo_off, hi_off):
                s0, s1 = seg(win)

                def ev(te, c):
                    rvec = sub_r[pl.ds(s0 + te * _L, _L)]
                    kvec = sub_k[pl.ds(s0 + te * _L, _L)]
                    off = rvec - r0
                    m = (off >= lo_off) & (off < hi_off)
                    for d1 in range(8):
                        x = plsc.load_gather(
                            chunk_v, [jnp.full((_L,), d1, jnp.int32), off],
                            mask=m)
                        plsc.store_scatter(
                            rows_v, [kvec * _D + (d0 * 8 + d1)], x, mask=m)
                    return c

                lax.fori_loop(0, _ceil16(s1 - s0), ev, 0)

            trip = jnp.where(w == _NBKT - 1, 8, _WIN)
            for d0 in range(8):
                def winbody(win, c, d0=d0):
                    r0 = pl.multiple_of(base + win * _RW, 128)
                    pltpu.sync_copy(tab_hbm.at[d0, :, pl.ds(r0, _RW)], chunk_v)
                    extract(d0, win, r0, _RW, 0, _RW)
                    return c

                lax.fori_loop(0, trip, winbody, 0)

                @pl.when(w == _NBKT - 1)
                def _tail_a(d0=d0):
                    pltpu.sync_copy(
                        tab_hbm.at[d0, :, pl.ds(_TAIL_A0, 512)],
                        chunk_v.at[:, pl.ds(0, 512)])
                    extract(d0, jnp.int32(8), jnp.int32(_TAIL_A0), 512, 0, 512)

            @pl.when(w == _NBKT - 1)
            def _tail_b():
                pltpu.sync_copy(tail_hbm, tail_v)
                s0, s1 = seg(jnp.int32(8))

                def tev(te, c):
                    rvec = sub_r[pl.ds(s0 + te * _L, _L)]
                    kvec = sub_k[pl.ds(s0 + te * _L, _L)]
                    off = rvec - _TAIL_R0
                    m = (off >= 0) & (off < _V - _TAIL_R0)
                    for col in range(_D):
                        x = plsc.load_gather(
                            tail_v, [jnp.full((_L,), col, jnp.int32), off],
                            mask=m)
                        plsc.store_scatter(rows_v, [kvec * _D + col], x,
                                           mask=m)
                    return c

                lax.fori_loop(0, _ceil16(s1 - s0), tev, 0)

            def norm(g, c):
                rows = (g * _L + lanes) * _D
                acc = [jnp.zeros((_L,), jnp.float32) for _ in range(4)]
                for col in range(_D):
                    x = plsc.load_gather(rows_v, [rows + col])
                    acc[col % 4] = acc[col % 4] + x * x
                ss = (acc[0] + acc[1]) + (acc[2] + acc[3])
                s = _rsqrt16(jnp.maximum(ss, 1e-24))
                for col in range(_D):
                    x = plsc.load_gather(rows_v, [rows + col])
                    plsc.store_scatter(rows_v, [rows + col], x * s)
                return c

            lax.fori_loop(0, _P // _L, norm, 0)

            bid = w * _MAXR + t
            pltpu.sync_copy(rows_v,
                            rows1_hbm.at[pl.ds(
                                pl.multiple_of(bid * (_P * _D), 1024),
                                _P * _D)])
            pltpu.sync_copy(pb_v.at[pl.ds(0, _P)],
                            bs_hbm.at[pl.ds(pl.multiple_of(bid * _P, 1024),
                                            _P)])
            return carry

        lax.fori_loop(0, rounds, round_body, 0)

        @pl.when(w < _NBKT)
        def _meta():
            for j in range(_P // _L):
                meta_v[pl.ds(j * _L, _L)] = jnp.zeros((_L,), jnp.int32) + count
            pltpu.sync_copy(meta_v,
                            meta_hbm.at[pl.ds(pl.multiple_of(w * _P, 1024),
                                              _P)])

    return pl.kernel(
        body,
        out_type=(
            jax.ShapeDtypeStruct((_NROWS1,), jnp.float32),
            jax.ShapeDtypeStruct((32 * _MAXR * _P,), jnp.int32),
            jax.ShapeDtypeStruct((32 * _P,), jnp.int32),
        ),
        mesh=mesh,
        compiler_params=pltpu.CompilerParams(needs_layout_passes=False),
        scratch_types=[
            pltpu.VMEM((4096,), jnp.int32),
            pltpu.VMEM((_P + _L,), jnp.int32),
            pltpu.VMEM((_P + _L,), jnp.int32),
            pltpu.VMEM((_P + _L,), jnp.int32),
            pltpu.VMEM((_P + _L,), jnp.int32),
            pltpu.VMEM(((_WIN + 1) * _L,), jnp.int32),
            pltpu.VMEM((8, _RW), jnp.float32),
            pltpu.VMEM((_P * _D,), jnp.float32),
            pltpu.VMEM((_P,), jnp.int32),
            pltpu.VMEM((_D, _V - _TAIL_R0), jnp.float32),
            pltpu.SemaphoreType.DMA,
        ],
    )


def _build_k2():
    mesh = plsc.VectorSubcoreMesh(**_MESH)

    def body(rows1_hbm, bs_hbm, meta_hbm, out_hbm,
             meta_v, q_v, pad_v, bs_v, bsq_v, sem):
        w = lax.axis_index("s") * _NC + lax.axis_index("c")

        @pl.when(w < _NBKT)
        def _work():
            pltpu.sync_copy(
                meta_hbm.at[pl.ds(pl.multiple_of(w * _P, 1024), _P)], meta_v)
            count = meta_v[pl.ds(0, _L)][0]
            blocks = (count + (_P - 1)) >> 10

            def blk(t, c):
                bid = w * _MAXR + t
                pltpu.sync_copy(
                    bs_hbm.at[pl.ds(pl.multiple_of(bid * _P, 1024), _P)], bs_v)
                for q in range(4):
                    pltpu.sync_copy(
                        rows1_hbm.at[pl.ds(
                            pl.multiple_of(bid * (_P * _D) + q * (256 * _D),
                                           1024), 256 * _D)], q_v)
                    for i in range(1024):
                        x = q_v[pl.ds(i * _L, _L)]
                        pad_v[i // 4, pl.ds((i % 4) * _L, _L)] = x
                    for i in range(256 // _L):
                        bsq_v[pl.ds(i * _L, _L)] = (
                            bs_v[pl.ds(q * 256 + i * _L, _L)])
                    pltpu.async_copy(pad_v, out_hbm.at[bsq_v], sem).wait()
                return c

            lax.fori_loop(0, blocks, blk, 0)

    return pl.kernel(
        body,
        out_type=jax.ShapeDtypeStruct((_B + _L, 128), jnp.float32),
        mesh=mesh,
        compiler_params=pltpu.CompilerParams(needs_layout_passes=False),
        scratch_types=[
            pltpu.VMEM((_P,), jnp.int32),
            pltpu.VMEM((256 * _D,), jnp.float32),
            pltpu.VMEM((256, 128), jnp.float32),
            pltpu.VMEM((_P,), jnp.int32),
            pltpu.VMEM((256,), jnp.int32),
            pltpu.SemaphoreType.DMA,
        ],
    )


def kernel(user_ids, table):
    ids = user_ids.astype(jnp.int32)
    tabf = table.astype(jnp.float32)
    tab3 = tabf.T.reshape(8, 8, _V)
    tail = tabf[_TAIL_R0:].T
    rows1, bs, meta = _build_k1()(ids, tab3, tail)
    out_pad = _build_k2()(rows1, bs, meta)
    return out_pad[:_B, :_D]


# speedup vs baseline: 1.0355x; 1.0355x over previous
import jax
import jax.numpy as jnp
from jax import lax
from jax.experimental import pallas as pl
from jax.experimental.pallas import tpu as pltpu
from jax.experimental.pallas import tpu_sc as plsc

_NC = 2
_NS = 16
_L = 16

_V = 1000000
_D = 64
_B = 16384

_P = 1024
_MAXR = 16
_RW = 2048
_NBKT = 31
_WIN = 16
_DUMP = _B

_TAIL_A0 = 999424
_TAIL_R0 = 999936

_NROWS1 = 32 * _MAXR * _P * _D


def _rsqrt16(x):
    i = plsc.bitcast(x, jnp.int32)
    y = plsc.bitcast(jnp.int32(0x5F3759DF) - (i >> 1), jnp.float32)
    for _ in range(3):
        y = y * (1.5 - (0.5 * x) * y * y)
    return y


def _ceil16(n):
    return (n + 15) >> 4


_MESH = dict(core_axis_name="c", subcore_axis_name="s",
             num_cores=_NC, num_subcores=_NS)


def _build_k1():
    mesh = plsc.VectorSubcoreMesh(**_MESH)

    def body(ids_hbm, tab_hbm, tail_hbm, rows1_hbm, bs_hbm, meta_hbm,
             idx_v, pr_v, pb_v, sub_r, sub_k, so_v, chunk_v, rows_v, meta_v,
             tail_v, sem):
        w = lax.axis_index("s") * _NC + lax.axis_index("c")
        lanes = lax.iota(jnp.int32, _L)
        base = w << 15

        def scan_ids(per_vreg):
            def outer(ci, o):
                pltpu.sync_copy(ids_hbm.at[pl.ds(ci * 4096, 4096)], idx_v)

                def blk(i, o):
                    rvec = idx_v[pl.ds(i * _L, _L)]
                    bvec = ci * 4096 + i * _L + lanes
                    return per_vreg(bvec, rvec, o)

                return lax.fori_loop(0, 256, blk, o)

            return lax.fori_loop(0, 4, outer, jnp.int32(0))

        def bucket_of(rvec):
            return jnp.minimum(rvec >> 15, jnp.int32(_NBKT - 1))

        def count_vreg(bvec, rvec, o):
            m = bucket_of(rvec) == w
            return o + plsc.all_reduce_population_count(m)[0]

        count = scan_ids(count_vreg)
        rounds = (count + (_P - 1)) >> 10

        def round_body(t, carry):
            lo = t * _P
            for j in range(_P // _L):
                pr_v[pl.ds(j * _L, _L)] = jnp.full((_L,), 0, jnp.int32) + base
                pb_v[pl.ds(j * _L, _L)] = jnp.full((_L,), _DUMP, jnp.int32)
            pr_v[pl.ds(_P, _L)] = jnp.full((_L,), 0, jnp.int32) + base
            pb_v[pl.ds(_P, _L)] = jnp.full((_L,), _DUMP, jnp.int32)

            def fill_vreg(bvec, rvec, o):
                m = bucket_of(rvec) == w
                cnt = plsc.all_reduce_population_count(m)[0]
                rank = o + plsc.cumsum(m.astype(jnp.int32)) - 1
                sel = m & (rank >= lo) & (rank < lo + _P)
                dst = jnp.minimum(jnp.maximum(o - lo, 0), _P)
                plsc.store_compressed(pr_v.at[pl.ds(dst, _L)], rvec, mask=sel)
                plsc.store_compressed(pb_v.at[pl.ds(dst, _L)], bvec, mask=sel)
                return o + cnt

            scan_ids(fill_vreg)

            o2 = jnp.int32(0)
            for win in range(_WIN):
                so_v[pl.ds(win * _L, _L)] = jnp.zeros((_L,), jnp.int32) + o2

                def sblk(i, o2, win=win):
                    rvec = pr_v[pl.ds(i * _L, _L)]
                    kvec = i * _L + lanes
                    m = ((rvec - base) >> 11) == win
                    cnt = plsc.all_reduce_population_count(m)[0]
                    plsc.store_compressed(sub_r.at[pl.ds(o2, _L)], rvec, mask=m)
                    plsc.store_compressed(sub_k.at[pl.ds(o2, _L)], kvec, mask=m)
                    return o2 + cnt

                o2 = lax.fori_loop(0, _P // _L, sblk, o2)
            so_v[pl.ds(_WIN * _L, _L)] = jnp.zeros((_L,), jnp.int32) + o2

            def seg(win):
                s0 = so_v[pl.ds(win * _L, _L)][0]
                s1 = so_v[pl.ds(win * _L + _L, _L)][0]
                return s0, s1

            def extract(d0, buf, win, r0, lo_off, hi_off):
                s0, s1 = seg(win)

                def ev(te, c):
                    rvec = sub_r[pl.ds(s0 + te * _L, _L)]
                    kvec = sub_k[pl.ds(s0 + te * _L, _L)]
                    off = rvec - r0
                    m = (off >= lo_off) & (off < hi_off)
                    for d1 in range(8):
                        x = plsc.load_gather(
                            chunk_v,
                            [buf, jnp.full((_L,), d1, jnp.int32), off],
                            mask=m)
                        plsc.store_scatter(
                            rows_v, [kvec * _D + (d0 * 8 + d1)], x, mask=m)
                    return c

                lax.fori_loop(0, _ceil16(s1 - s0), ev, 0)

            def issue(d0, win, slot):
                r0 = pl.multiple_of(base + win * _RW, 128)
                pltpu.async_copy(tab_hbm.at[d0, :, pl.ds(r0, _RW)],
                                 chunk_v.at[slot], sem)

            def drain(slot):
                pltpu.make_async_copy(tab_hbm.at[0, :, pl.ds(0, _RW)],
                                      chunk_v.at[slot], sem).wait()

            trip = jnp.where(w == _NBKT - 1, 8, _WIN)
            for d0 in range(8):
                issue(d0, jnp.int32(0), jnp.int32(0))

                def winbody(win, c, d0=d0):
                    slot = win & 1
                    drain(slot)

                    @pl.when(win + 1 < trip)
                    def _next():
                        issue(d0, win + 1, (win + 1) & 1)

                    r0 = pl.multiple_of(base + win * _RW, 128)
                    bufv = jnp.full((_L,), 0, jnp.int32) + slot
                    extract(d0, bufv, win, r0, 0, _RW)
                    return c

                lax.fori_loop(0, trip, winbody, 0)

                @pl.when(w == _NBKT - 1)
                def _tail_a(d0=d0):
                    pltpu.sync_copy(
                        tab_hbm.at[d0, :, pl.ds(_TAIL_A0, 512)],
                        chunk_v.at[0, :, pl.ds(0, 512)])
                    extract(d0, jnp.full((_L,), 0, jnp.int32), jnp.int32(8),
                            jnp.int32(_TAIL_A0), 0, 512)

            @pl.when(w == _NBKT - 1)
            def _tail_b():
                pltpu.sync_copy(tail_hbm, tail_v)
                s0, s1 = seg(jnp.int32(8))

                def tev(te, c):
                    rvec = sub_r[pl.ds(s0 + te * _L, _L)]
                    kvec = sub_k[pl.ds(s0 + te * _L, _L)]
                    off = rvec - _TAIL_R0
                    m = (off >= 0) & (off < _V - _TAIL_R0)
                    for col in range(_D):
                        x = plsc.load_gather(
                            tail_v, [jnp.full((_L,), col, jnp.int32), off],
                            mask=m)
                        plsc.store_scatter(rows_v, [kvec * _D + col], x,
                                           mask=m)
                    return c

                lax.fori_loop(0, _ceil16(s1 - s0), tev, 0)

            def norm(g, c):
                rows = (g * _L + lanes) * _D
                acc = [jnp.zeros((_L,), jnp.float32) for _ in range(4)]
                for col in range(_D):
                    x = plsc.load_gather(rows_v, [rows + col])
                    acc[col % 4] = acc[col % 4] + x * x
                ss = (acc[0] + acc[1]) + (acc[2] + acc[3])
                s = _rsqrt16(jnp.maximum(ss, 1e-24))
                for col in range(_D):
                    x = plsc.load_gather(rows_v, [rows + col])
                    plsc.store_scatter(rows_v, [rows + col], x * s)
                return c

            lax.fori_loop(0, _P // _L, norm, 0)

            bid = w * _MAXR + t
            pltpu.sync_copy(rows_v,
                            rows1_hbm.at[pl.ds(
                                pl.multiple_of(bid * (_P * _D), 1024),
                                _P * _D)])
            pltpu.sync_copy(pb_v.at[pl.ds(0, _P)],
                            bs_hbm.at[pl.ds(pl.multiple_of(bid * _P, 1024),
                                            _P)])
            return carry

        lax.fori_loop(0, rounds, round_body, 0)

        @pl.when(w < _NBKT)
        def _meta():
            for j in range(_P // _L):
                meta_v[pl.ds(j * _L, _L)] = jnp.zeros((_L,), jnp.int32) + count
            pltpu.sync_copy(meta_v,
                            meta_hbm.at[pl.ds(pl.multiple_of(w * _P, 1024),
                                              _P)])

    return pl.kernel(
        body,
        out_type=(
            jax.ShapeDtypeStruct((_NROWS1,), jnp.float32),
            jax.ShapeDtypeStruct((32 * _MAXR * _P,), jnp.int32),
            jax.ShapeDtypeStruct((32 * _P,), jnp.int32),
        ),
        mesh=mesh,
        compiler_params=pltpu.CompilerParams(needs_layout_passes=False),
        scratch_types=[
            pltpu.VMEM((4096,), jnp.int32),
            pltpu.VMEM((_P + _L,), jnp.int32),
            pltpu.VMEM((_P + _L,), jnp.int32),
            pltpu.VMEM((_P + _L,), jnp.int32),
            pltpu.VMEM((_P + _L,), jnp.int32),
            pltpu.VMEM(((_WIN + 1) * _L,), jnp.int32),
            pltpu.VMEM((2, 8, _RW), jnp.float32),
            pltpu.VMEM((_P * _D,), jnp.float32),
            pltpu.VMEM((_P,), jnp.int32),
            pltpu.VMEM((_D, _V - _TAIL_R0), jnp.float32),
            pltpu.SemaphoreType.DMA,
        ],
    )


def _build_k2():
    mesh = plsc.VectorSubcoreMesh(**_MESH)

    def body(rows1_hbm, bs_hbm, meta_hbm, out_hbm,
             meta_v, q_v, pad_v, bs_v, bsq_v, sem):
        w = lax.axis_index("s") * _NC + lax.axis_index("c")

        @pl.when(w < _NBKT)
        def _work():
            pltpu.sync_copy(
                meta_hbm.at[pl.ds(pl.multiple_of(w * _P, 1024), _P)], meta_v)
            count = meta_v[pl.ds(0, _L)][0]
            blocks = (count + (_P - 1)) >> 10

            def blk(t, c):
                bid = w * _MAXR + t
                pltpu.sync_copy(
                    bs_hbm.at[pl.ds(pl.multiple_of(bid * _P, 1024), _P)], bs_v)
                for q in range(4):
                    pltpu.sync_copy(
                        rows1_hbm.at[pl.ds(
                            pl.multiple_of(bid * (_P * _D) + q * (256 * _D),
                                           1024), 256 * _D)], q_v)
                    def expand(i, c2):
                        for j in range(4):
                            x = q_v[pl.ds((i * 4 + j) * _L, _L)]
                            pad_v[i, pl.ds(j * _L, _L)] = x
                        return c2

                    lax.fori_loop(0, 256, expand, 0)
                    for i in range(256 // _L):
                        bsq_v[pl.ds(i * _L, _L)] = (
                            bs_v[pl.ds(q * 256 + i * _L, _L)])
                    pltpu.async_copy(pad_v, out_hbm.at[bsq_v], sem).wait()
                return c

            lax.fori_loop(0, blocks, blk, 0)

    return pl.kernel(
        body,
        out_type=jax.ShapeDtypeStruct((_B + _L, 128), jnp.float32),
        mesh=mesh,
        compiler_params=pltpu.CompilerParams(needs_layout_passes=False),
        scratch_types=[
            pltpu.VMEM((_P,), jnp.int32),
            pltpu.VMEM((256 * _D,), jnp.float32),
            pltpu.VMEM((256, 128), jnp.float32),
            pltpu.VMEM((_P,), jnp.int32),
            pltpu.VMEM((256,), jnp.int32),
            pltpu.SemaphoreType.DMA,
        ],
    )


def kernel(user_ids, table):
    ids = user_ids.astype(jnp.int32)
    tabf = table.astype(jnp.float32)
    tab3 = tabf.T.reshape(8, 8, _V)
    tail = tabf[_TAIL_R0:].T
    rows1, bs, meta = _build_k1()(ids, tab3, tail)
    out_pad = _build_k2()(rows1, bs, meta)
    return out_pad[:_B, :_D]


# speedup vs baseline: 2.5767x; 2.4883x over previous
import jax
import jax.numpy as jnp
from jax import lax
from jax.experimental import pallas as pl
from jax.experimental.pallas import tpu as pltpu
from jax.experimental.pallas import tpu_sc as plsc

_NC = 2
_NS = 16
_L = 16

_V = 1000000
_D = 64
_B = 16384

_P = 1024
_MAXR = 16
_RW = 2048
_NBKT = 31
_WIN = 16
_DUMP = _B
_NDUMP = 512

_TAIL_A0 = 999424
_TAIL_R0 = 999936

_NROWS1 = 32 * _MAXR * _P * _D


def _rsqrt16(x):
    i = plsc.bitcast(x, jnp.int32)
    y = plsc.bitcast(jnp.int32(0x5F3759DF) - (i >> 1), jnp.float32)
    for _ in range(3):
        y = y * (1.5 - (0.5 * x) * y * y)
    return y


def _ceil16(n):
    return (n + 15) >> 4


_MESH = dict(core_axis_name="c", subcore_axis_name="s",
             num_cores=_NC, num_subcores=_NS)


def _build_k1():
    mesh = plsc.VectorSubcoreMesh(**_MESH)

    def body(ids_hbm, tab_hbm, tail_hbm, rows1_hbm, bs_hbm, meta_hbm,
             idx_v, pr_v, pb_v, sub_r, sub_k, so_v, chunk_v, rows_v, meta_v,
             tail_v, sem):
        w = lax.axis_index("s") * _NC + lax.axis_index("c")
        lanes = lax.iota(jnp.int32, _L)
        base = w << 15

        def scan_ids(per_vreg):
            def outer(ci, o):
                pltpu.sync_copy(ids_hbm.at[pl.ds(ci * 4096, 4096)], idx_v)

                def blk(i, o):
                    rvec = idx_v[pl.ds(i * _L, _L)]
                    bvec = ci * 4096 + i * _L + lanes
                    return per_vreg(bvec, rvec, o)

                return lax.fori_loop(0, 256, blk, o)

            return lax.fori_loop(0, 4, outer, jnp.int32(0))

        def bucket_of(rvec):
            return jnp.minimum(rvec >> 15, jnp.int32(_NBKT - 1))

        def count_vreg(bvec, rvec, o):
            m = bucket_of(rvec) == w
            return o + plsc.all_reduce_population_count(m)[0]

        count = scan_ids(count_vreg)
        rounds = (count + (_P - 1)) >> 10

        def round_body(t, carry):
            lo = t * _P
            for j in range(_P // _L + 1):
                dump = _DUMP + ((w * 17 + j * _L + lanes) & (_NDUMP - 1))
                pr_v[pl.ds(j * _L, _L)] = jnp.full((_L,), 0, jnp.int32) + base
                pb_v[pl.ds(j * _L, _L)] = dump

            def fill_vreg(bvec, rvec, o):
                m = bucket_of(rvec) == w
                cnt = plsc.all_reduce_population_count(m)[0]
                rank = o + plsc.cumsum(m.astype(jnp.int32)) - 1
                sel = m & (rank >= lo) & (rank < lo + _P)
                dst = jnp.minimum(jnp.maximum(o - lo, 0), _P)
                plsc.store_compressed(pr_v.at[pl.ds(dst, _L)], rvec, mask=sel)
                plsc.store_compressed(pb_v.at[pl.ds(dst, _L)], bvec, mask=sel)
                return o + cnt

            scan_ids(fill_vreg)

            o2 = jnp.int32(0)
            for win in range(_WIN):
                so_v[pl.ds(win * _L, _L)] = jnp.zeros((_L,), jnp.int32) + o2

                def sblk(i, o2, win=win):
                    rvec = pr_v[pl.ds(i * _L, _L)]
                    kvec = i * _L + lanes
                    m = ((rvec - base) >> 11) == win
                    cnt = plsc.all_reduce_population_count(m)[0]
                    plsc.store_compressed(sub_r.at[pl.ds(o2, _L)], rvec, mask=m)
                    plsc.store_compressed(sub_k.at[pl.ds(o2, _L)], kvec, mask=m)
                    return o2 + cnt

                o2 = lax.fori_loop(0, _P // _L, sblk, o2)
            so_v[pl.ds(_WIN * _L, _L)] = jnp.zeros((_L,), jnp.int32) + o2

            def seg(win):
                s0 = so_v[pl.ds(win * _L, _L)][0]
                s1 = so_v[pl.ds(win * _L + _L, _L)][0]
                return s0, s1

            def extract(d0, buf, win, r0, lo_off, hi_off):
                s0, s1 = seg(win)

                def ev(te, c):
                    rvec = sub_r[pl.ds(s0 + te * _L, _L)]
                    kvec = sub_k[pl.ds(s0 + te * _L, _L)]
                    off = rvec - r0
                    m = (off >= lo_off) & (off < hi_off)
                    for d1 in range(8):
                        x = plsc.load_gather(
                            chunk_v,
                            [buf, jnp.full((_L,), d1, jnp.int32), off],
                            mask=m)
                        plsc.store_scatter(
                            rows_v, [kvec * _D + (d0 * 8 + d1)], x, mask=m)
                    return c

                lax.fori_loop(0, _ceil16(s1 - s0), ev, 0)

            def issue(d0, win, slot):
                r0 = pl.multiple_of(base + win * _RW, 128)
                pltpu.async_copy(tab_hbm.at[d0, :, pl.ds(r0, _RW)],
                                 chunk_v.at[slot], sem)

            def drain(slot):
                pltpu.make_async_copy(tab_hbm.at[0, :, pl.ds(0, _RW)],
                                      chunk_v.at[slot], sem).wait()

            trip = jnp.where(w == _NBKT - 1, 8, _WIN)
            for d0 in range(8):
                issue(d0, jnp.int32(0), jnp.int32(0))

                def winbody(win, c, d0=d0):
                    slot = win & 1
                    drain(slot)

                    @pl.when(win + 1 < trip)
                    def _next():
                        issue(d0, win + 1, (win + 1) & 1)

                    r0 = pl.multiple_of(base + win * _RW, 128)
                    bufv = jnp.full((_L,), 0, jnp.int32) + slot
                    extract(d0, bufv, win, r0, 0, _RW)
                    return c

                lax.fori_loop(0, trip, winbody, 0)

                @pl.when(w == _NBKT - 1)
                def _tail_a(d0=d0):
                    pltpu.sync_copy(
                        tab_hbm.at[d0, :, pl.ds(_TAIL_A0, 512)],
                        chunk_v.at[0, :, pl.ds(0, 512)])
                    extract(d0, jnp.full((_L,), 0, jnp.int32), jnp.int32(8),
                            jnp.int32(_TAIL_A0), 0, 512)

            @pl.when(w == _NBKT - 1)
            def _tail_b():
                pltpu.sync_copy(tail_hbm, tail_v)
                s0, s1 = seg(jnp.int32(8))

                def tev(te, c):
                    rvec = sub_r[pl.ds(s0 + te * _L, _L)]
                    kvec = sub_k[pl.ds(s0 + te * _L, _L)]
                    off = rvec - _TAIL_R0
                    m = (off >= 0) & (off < _V - _TAIL_R0)
                    for col in range(_D):
                        x = plsc.load_gather(
                            tail_v, [jnp.full((_L,), col, jnp.int32), off],
                            mask=m)
                        plsc.store_scatter(rows_v, [kvec * _D + col], x,
                                           mask=m)
                    return c

                lax.fori_loop(0, _ceil16(s1 - s0), tev, 0)

            def norm(g, c):
                rows = (g * _L + lanes) * _D
                acc = [jnp.zeros((_L,), jnp.float32) for _ in range(4)]
                for col in range(_D):
                    x = plsc.load_gather(rows_v, [rows + col])
                    acc[col % 4] = acc[col % 4] + x * x
                ss = (acc[0] + acc[1]) + (acc[2] + acc[3])
                s = _rsqrt16(jnp.maximum(ss, 1e-24))
                for col in range(_D):
                    x = plsc.load_gather(rows_v, [rows + col])
                    plsc.store_scatter(rows_v, [rows + col], x * s)
                return c

            lax.fori_loop(0, _P // _L, norm, 0)

            bid = w * _MAXR + t
            pltpu.sync_copy(rows_v,
                            rows1_hbm.at[pl.ds(
                                pl.multiple_of(bid * (_P * _D), 1024),
                                _P * _D)])
            pltpu.sync_copy(pb_v.at[pl.ds(0, _P)],
                            bs_hbm.at[pl.ds(pl.multiple_of(bid * _P, 1024),
                                            _P)])
            return carry

        lax.fori_loop(0, rounds, round_body, 0)

        @pl.when(w < _NBKT)
        def _meta():
            for j in range(_P // _L):
                meta_v[pl.ds(j * _L, _L)] = jnp.zeros((_L,), jnp.int32) + count
            pltpu.sync_copy(meta_v,
                            meta_hbm.at[pl.ds(pl.multiple_of(w * _P, 1024),
                                              _P)])

    return pl.kernel(
        body,
        out_type=(
            jax.ShapeDtypeStruct((_NROWS1,), jnp.float32),
            jax.ShapeDtypeStruct((32 * _MAXR * _P,), jnp.int32),
            jax.ShapeDtypeStruct((32 * _P,), jnp.int32),
        ),
        mesh=mesh,
        compiler_params=pltpu.CompilerParams(needs_layout_passes=False),
        scratch_types=[
            pltpu.VMEM((4096,), jnp.int32),
            pltpu.VMEM((_P + _L,), jnp.int32),
            pltpu.VMEM((_P + _L,), jnp.int32),
            pltpu.VMEM((_P + _L,), jnp.int32),
            pltpu.VMEM((_P + _L,), jnp.int32),
            pltpu.VMEM(((_WIN + 1) * _L,), jnp.int32),
            pltpu.VMEM((2, 8, _RW), jnp.float32),
            pltpu.VMEM((_P * _D,), jnp.float32),
            pltpu.VMEM((_P,), jnp.int32),
            pltpu.VMEM((_D, _V - _TAIL_R0), jnp.float32),
            pltpu.SemaphoreType.DMA,
        ],
    )


def _build_k2():
    mesh = plsc.VectorSubcoreMesh(**_MESH)

    def body(rows1_hbm, bs_hbm, meta_hbm, out_hbm,
             meta_v, q_v, pad_v, bs_v, bsq_v, sem):
        w = lax.axis_index("s") * _NC + lax.axis_index("c")

        @pl.when(w < _NBKT)
        def _work():
            pltpu.sync_copy(
                meta_hbm.at[pl.ds(pl.multiple_of(w * _P, 1024), _P)], meta_v)
            count = meta_v[pl.ds(0, _L)][0]
            blocks = (count + (_P - 1)) >> 10

            def blk(t, c):
                bid = w * _MAXR + t
                rem = jnp.minimum(count - t * _P, _P)
                nq = (rem + 255) >> 8
                pltpu.sync_copy(
                    bs_hbm.at[pl.ds(pl.multiple_of(bid * _P, 1024), _P)], bs_v)

                def quarter(q, c2):
                    pltpu.sync_copy(
                        rows1_hbm.at[pl.ds(
                            pl.multiple_of(bid * (_P * _D) + q * (256 * _D),
                                           1024), 256 * _D)], q_v)

                    def expand(i, c3):
                        for j in range(4):
                            x = q_v[pl.ds((i * 4 + j) * _L, _L)]
                            pad_v[i, pl.ds(j * _L, _L)] = x
                        return c3

                    lax.fori_loop(0, 256, expand, 0)

                    def bscp(i, c3):
                        bsq_v[pl.ds(i * _L, _L)] = (
                            bs_v[pl.ds(q * 256 + i * _L, _L)])
                        return c3

                    lax.fori_loop(0, 256 // _L, bscp, 0)
                    pltpu.async_copy(pad_v, out_hbm.at[bsq_v], sem).wait()
                    return c2

                lax.fori_loop(0, nq, quarter, 0)
                return c

            lax.fori_loop(0, blocks, blk, 0)

    return pl.kernel(
        body,
        out_type=jax.ShapeDtypeStruct((_B + _NDUMP, 128), jnp.float32),
        mesh=mesh,
        compiler_params=pltpu.CompilerParams(needs_layout_passes=False),
        scratch_types=[
            pltpu.VMEM((_P,), jnp.int32),
            pltpu.VMEM((256 * _D,), jnp.float32),
            pltpu.VMEM((256, 128), jnp.float32),
            pltpu.VMEM((_P,), jnp.int32),
            pltpu.VMEM((256,), jnp.int32),
            pltpu.SemaphoreType.DMA,
        ],
    )


def kernel(user_ids, table):
    ids = user_ids.astype(jnp.int32)
    tabf = table.astype(jnp.float32)
    tab3 = tabf.T.reshape(8, 8, _V)
    tail = tabf[_TAIL_R0:].T
    rows1, bs, meta = _build_k1()(ids, tab3, tail)
    out_pad = _build_k2()(rows1, bs, meta)
    return out_pad[:_B, :_D]


# speedup vs baseline: 2.6296x; 1.0205x over previous
import jax
import jax.numpy as jnp
from jax import lax
from jax.experimental import pallas as pl
from jax.experimental.pallas import tpu as pltpu
from jax.experimental.pallas import tpu_sc as plsc

_NC = 2
_NS = 16
_L = 16

_V = 1000000
_D = 64
_B = 16384

_P = 1024
_MAXR = 16
_RW = 2048
_NBKT = 31
_WIN = 16
_DUMP = _B
_NDUMP = 512

_TAIL_A0 = 999424
_TAIL_R0 = 999936

_NROWS1 = 32 * _MAXR * _P * _D


def _rsqrt16(x):
    i = plsc.bitcast(x, jnp.int32)
    y = plsc.bitcast(jnp.int32(0x5F3759DF) - (i >> 1), jnp.float32)
    for _ in range(3):
        y = y * (1.5 - (0.5 * x) * y * y)
    return y


def _ceil16(n):
    return (n + 15) >> 4


_MESH = dict(core_axis_name="c", subcore_axis_name="s",
             num_cores=_NC, num_subcores=_NS)


def _build_k1():
    mesh = plsc.VectorSubcoreMesh(**_MESH)

    def body(ids_hbm, tab_hbm, tail_hbm, rows1_hbm, bs_hbm, meta_hbm,
             idx_v, pr_v, pb_v, sub_r, sub_k, so_v, chunk_v, rows_v, meta_v,
             tail_v, sem):
        w = lax.axis_index("s") * _NC + lax.axis_index("c")
        lanes = lax.iota(jnp.int32, _L)
        base = w << 15

        def scan_ids(per_vreg):
            def outer(ci, o):
                pltpu.sync_copy(ids_hbm.at[pl.ds(ci * 4096, 4096)], idx_v)

                def blk(i, o):
                    rvec = idx_v[pl.ds(i * _L, _L)]
                    bvec = ci * 4096 + i * _L + lanes
                    return per_vreg(bvec, rvec, o)

                return lax.fori_loop(0, 256, blk, o)

            return lax.fori_loop(0, 4, outer, jnp.int32(0))

        def bucket_of(rvec):
            return jnp.minimum(rvec >> 15, jnp.int32(_NBKT - 1))

        def round_body(carry):
            t, _prev = carry
            lo = t * _P
            for j in range(_P // _L + 1):
                dump = _DUMP + ((w * 17 + j * _L + lanes) & (_NDUMP - 1))
                pr_v[pl.ds(j * _L, _L)] = jnp.full((_L,), 0, jnp.int32) + base
                pb_v[pl.ds(j * _L, _L)] = dump

            def fill_vreg(bvec, rvec, o):
                m = bucket_of(rvec) == w
                cnt = plsc.all_reduce_population_count(m)[0]
                rank = o + plsc.cumsum(m.astype(jnp.int32)) - 1
                sel = m & (rank >= lo) & (rank < lo + _P)
                dst = jnp.minimum(jnp.maximum(o - lo, 0), _P)
                plsc.store_compressed(pr_v.at[pl.ds(dst, _L)], rvec, mask=sel)
                plsc.store_compressed(pb_v.at[pl.ds(dst, _L)], bvec, mask=sel)
                return o + cnt

            count = scan_ids(fill_vreg)

            o2 = jnp.int32(0)
            for win in range(_WIN):
                so_v[pl.ds(win * _L, _L)] = jnp.zeros((_L,), jnp.int32) + o2

                def sblk(i, o2, win=win):
                    rvec = pr_v[pl.ds(i * _L, _L)]
                    kvec = i * _L + lanes
                    m = ((rvec - base) >> 11) == win
                    cnt = plsc.all_reduce_population_count(m)[0]
                    plsc.store_compressed(sub_r.at[pl.ds(o2, _L)], rvec, mask=m)
                    plsc.store_compressed(sub_k.at[pl.ds(o2, _L)], kvec, mask=m)
                    return o2 + cnt

                o2 = lax.fori_loop(0, _P // _L, sblk, o2)
            so_v[pl.ds(_WIN * _L, _L)] = jnp.zeros((_L,), jnp.int32) + o2

            def seg(win):
                s0 = so_v[pl.ds(win * _L, _L)][0]
                s1 = so_v[pl.ds(win * _L + _L, _L)][0]
                return s0, s1

            def extract(d0, buf, win, r0, lo_off, hi_off):
                s0, s1 = seg(win)

                def ev(te, c):
                    rvec = sub_r[pl.ds(s0 + te * _L, _L)]
                    kvec = sub_k[pl.ds(s0 + te * _L, _L)]
                    off = rvec - r0
                    m = (off >= lo_off) & (off < hi_off)
                    for d1 in range(8):
                        x = plsc.load_gather(
                            chunk_v,
                            [buf, jnp.full((_L,), d1, jnp.int32), off],
                            mask=m)
                        plsc.store_scatter(
                            rows_v, [kvec * _D + (d0 * 8 + d1)], x, mask=m)
                    return c

                lax.fori_loop(0, _ceil16(s1 - s0), ev, 0)

            def issue(d0, win, slot):
                r0 = pl.multiple_of(base + win * _RW, 128)
                pltpu.async_copy(tab_hbm.at[d0, :, pl.ds(r0, _RW)],
                                 chunk_v.at[slot], sem)

            def drain(slot):
                pltpu.make_async_copy(tab_hbm.at[0, :, pl.ds(0, _RW)],
                                      chunk_v.at[slot], sem).wait()

            trip = jnp.where(w == _NBKT - 1, 8, _WIN)
            for d0 in range(8):
                issue(d0, jnp.int32(0), jnp.int32(0))

                def winbody(win, c, d0=d0):
                    slot = win & 1
                    drain(slot)

                    @pl.when(win + 1 < trip)
                    def _next():
                        issue(d0, win + 1, (win + 1) & 1)

                    r0 = pl.multiple_of(base + win * _RW, 128)
                    bufv = jnp.full((_L,), 0, jnp.int32) + slot
                    extract(d0, bufv, win, r0, 0, _RW)
                    return c

                lax.fori_loop(0, trip, winbody, 0)

                @pl.when(w == _NBKT - 1)
                def _tail_a(d0=d0):
                    pltpu.sync_copy(
                        tab_hbm.at[d0, :, pl.ds(_TAIL_A0, 512)],
                        chunk_v.at[0, :, pl.ds(0, 512)])
                    extract(d0, jnp.full((_L,), 0, jnp.int32), jnp.int32(8),
                            jnp.int32(_TAIL_A0), 0, 512)

            @pl.when(w == _NBKT - 1)
            def _tail_b():
                pltpu.sync_copy(tail_hbm, tail_v)
                s0, s1 = seg(jnp.int32(8))

                def tev(te, c):
                    rvec = sub_r[pl.ds(s0 + te * _L, _L)]
                    kvec = sub_k[pl.ds(s0 + te * _L, _L)]
                    off = rvec - _TAIL_R0
                    m = (off >= 0) & (off < _V - _TAIL_R0)
                    for col in range(_D):
                        x = plsc.load_gather(
                            tail_v, [jnp.full((_L,), col, jnp.int32), off],
                            mask=m)
                        plsc.store_scatter(rows_v, [kvec * _D + col], x,
                                           mask=m)
                    return c

                lax.fori_loop(0, _ceil16(s1 - s0), tev, 0)

            def norm(g, c):
                rows = (g * _L + lanes) * _D
                acc = [jnp.zeros((_L,), jnp.float32) for _ in range(4)]
                for col in range(_D):
                    x = plsc.load_gather(rows_v, [rows + col])
                    acc[col % 4] = acc[col % 4] + x * x
                ss = (acc[0] + acc[1]) + (acc[2] + acc[3])
                s = _rsqrt16(jnp.maximum(ss, 1e-24))
                for col in range(_D):
                    x = plsc.load_gather(rows_v, [rows + col])
                    plsc.store_scatter(rows_v, [rows + col], x * s)
                return c

            lax.fori_loop(0, _P // _L, norm, 0)

            bid = w * _MAXR + t
            pltpu.sync_copy(rows_v,
                            rows1_hbm.at[pl.ds(
                                pl.multiple_of(bid * (_P * _D), 1024),
                                _P * _D)])
            pltpu.sync_copy(pb_v.at[pl.ds(0, _P)],
                            bs_hbm.at[pl.ds(pl.multiple_of(bid * _P, 1024),
                                            _P)])
            return (t + 1, count)

        def round_cond(carry):
            t, c = carry
            return ((t == 0) & (w < _NBKT)) | (t * _P < c)

        _, count = lax.while_loop(round_cond, round_body,
                                  (jnp.int32(0), jnp.int32(0)))

        @pl.when(w < _NBKT)
        def _meta():
            for j in range(_P // _L):
                meta_v[pl.ds(j * _L, _L)] = jnp.zeros((_L,), jnp.int32) + count
            pltpu.sync_copy(meta_v,
                            meta_hbm.at[pl.ds(pl.multiple_of(w * _P, 1024),
                                              _P)])

    return pl.kernel(
        body,
        out_type=(
            jax.ShapeDtypeStruct((_NROWS1,), jnp.float32),
            jax.ShapeDtypeStruct((32 * _MAXR * _P,), jnp.int32),
            jax.ShapeDtypeStruct((32 * _P,), jnp.int32),
        ),
        mesh=mesh,
        compiler_params=pltpu.CompilerParams(needs_layout_passes=False),
        scratch_types=[
            pltpu.VMEM((4096,), jnp.int32),
            pltpu.VMEM((_P + _L,), jnp.int32),
            pltpu.VMEM((_P + _L,), jnp.int32),
            pltpu.VMEM((_P + _L,), jnp.int32),
            pltpu.VMEM((_P + _L,), jnp.int32),
            pltpu.VMEM(((_WIN + 1) * _L,), jnp.int32),
            pltpu.VMEM((2, 8, _RW), jnp.float32),
            pltpu.VMEM((_P * _D,), jnp.float32),
            pltpu.VMEM((_P,), jnp.int32),
            pltpu.VMEM((_D, _V - _TAIL_R0), jnp.float32),
            pltpu.SemaphoreType.DMA,
        ],
    )


def _build_k2():
    mesh = plsc.VectorSubcoreMesh(**_MESH)

    def body(rows1_hbm, bs_hbm, meta_hbm, out_hbm,
             meta_v, q_v, pad_v, bs_v, bsq_v, sem):
        w = lax.axis_index("s") * _NC + lax.axis_index("c")

        @pl.when(w < _NBKT)
        def _work():
            pltpu.sync_copy(
                meta_hbm.at[pl.ds(pl.multiple_of(w * _P, 1024), _P)], meta_v)
            count = meta_v[pl.ds(0, _L)][0]
            blocks = (count + (_P - 1)) >> 10

            def blk(t, c):
                bid = w * _MAXR + t
                rem = jnp.minimum(count - t * _P, _P)
                nq = (rem + 255) >> 8
                pltpu.sync_copy(
                    bs_hbm.at[pl.ds(pl.multiple_of(bid * _P, 1024), _P)], bs_v)

                def quarter(q, c2):
                    pltpu.sync_copy(
                        rows1_hbm.at[pl.ds(
                            pl.multiple_of(bid * (_P * _D) + q * (256 * _D),
                                           1024), 256 * _D)], q_v)

                    def expand(i, c3):
                        for j in range(4):
                            x = q_v[pl.ds((i * 4 + j) * _L, _L)]
                            pad_v[i, pl.ds(j * _L, _L)] = x
                        return c3

                    lax.fori_loop(0, 256, expand, 0)

                    def bscp(i, c3):
                        bsq_v[pl.ds(i * _L, _L)] = (
                            bs_v[pl.ds(q * 256 + i * _L, _L)])
                        return c3

                    lax.fori_loop(0, 256 // _L, bscp, 0)
                    pltpu.async_copy(pad_v, out_hbm.at[bsq_v], sem).wait()
                    return c2

                lax.fori_loop(0, nq, quarter, 0)
                return c

            lax.fori_loop(0, blocks, blk, 0)

    return pl.kernel(
        body,
        out_type=jax.ShapeDtypeStruct((_B + _NDUMP, 128), jnp.float32),
        mesh=mesh,
        compiler_params=pltpu.CompilerParams(needs_layout_passes=False),
        scratch_types=[
            pltpu.VMEM((_P,), jnp.int32),
            pltpu.VMEM((256 * _D,), jnp.float32),
            pltpu.VMEM((256, 128), jnp.float32),
            pltpu.VMEM((_P,), jnp.int32),
            pltpu.VMEM((256,), jnp.int32),
            pltpu.SemaphoreType.DMA,
        ],
    )


def kernel(user_ids, table):
    ids = user_ids.astype(jnp.int32)
    tabf = table.astype(jnp.float32)
    tab3 = tabf.T.reshape(8, 8, _V)
    tail = tabf[_TAIL_R0:].T
    rows1, bs, meta = _build_k1()(ids, tab3, tail)
    out_pad = _build_k2()(rows1, bs, meta)
    return out_pad[:_B, :_D]
